# baseline (speedup 1.0000x reference)
import functools
import jax
import jax.numpy as jnp
from jax import lax
from jax.experimental import pallas as pl
from jax.experimental.pallas import tpu as pltpu
from jax.experimental.pallas import tpu_sc as plsc

B = 16384
D = 64
BLK = 2048
GRID = B // BLK

NV = 1000000
VCH = 8192
VGRID = (NV + VCH - 1) // VCH


def _tr_body(ent_t, ae_t, ent_o, ae_o):
    eye = jax.lax.broadcasted_iota(jnp.int32, (D, D), 0) == \
          jax.lax.broadcasted_iota(jnp.int32, (D, D), 1)
    ident = eye.astype(jnp.bfloat16)
    dn = (((0,), (0,)), ((), ()))
    def pack(y):
        u = lax.bitcast_convert_type(y, jnp.uint32) + jnp.uint32(0x8000)
        lo = u[:, : D // 2] >> jnp.uint32(16)
        hi = u[:, D // 2:] & jnp.uint32(0xFFFF0000)
        return lax.bitcast_convert_type(hi | lo, jnp.int32)

    ent_o[...] = pack(lax.dot_general(ent_t[...].astype(jnp.bfloat16), ident, dn,
                                      preferred_element_type=jnp.float32))
    ae_o[...] = pack(lax.dot_general(ae_t[...].astype(jnp.bfloat16), ident, dn,
                                     preferred_element_type=jnp.float32))


def _tc_transpose(ent_t, ae_t):
    col = pl.BlockSpec((D, VCH), lambda i: (0, i))
    row = pl.BlockSpec((VCH, D // 2), lambda i: (i, 0))
    return pl.pallas_call(
        _tr_body,
        grid=(VGRID,),
        in_specs=[col, col],
        out_specs=[row, row],
        out_shape=[
            jax.ShapeDtypeStruct((NV, D // 2), jnp.int32),
            jax.ShapeDtypeStruct((NV, D // 2), jnp.int32),
        ],
    )(ent_t, ae_t)


@functools.cache
def _make_sc_gather():
    info = plsc.get_sparse_core_info()
    NC, NS = info.num_cores, info.num_subcores
    NW = NC * NS
    bpw = B // NW
    UNROLL = 16
    CH = bpw // 2

    mesh = plsc.VectorSubcoreMesh(core_axis_name="c", subcore_axis_name="s")
    outb = jax.ShapeDtypeStruct((B, D // 2), jnp.int32)
    outf = jax.ShapeDtypeStruct((B, D), jnp.float32)

    @functools.partial(
        pl.kernel,
        mesh=mesh,
        out_type=[outb] * 4 + [outf] * 2,
        scratch_types=[
            pltpu.VMEM((bpw,), jnp.int32),
            pltpu.VMEM((CH, D // 2), jnp.int32),
            pltpu.VMEM((CH, D // 2), jnp.int32),
            pltpu.VMEM((CH, D), jnp.float32),
            pltpu.SemaphoreType.DMA,
            pltpu.SemaphoreType.DMA,
        ],
    )
    def sc_gather(heads, tails, rels, ent, ae, rel_t, rinv_t,
                  hh_o, tt_o, hx_o, tx_o, r_o, rinv_o,
                  idx_v, rows_a, rows_b, rows_fa, sem_a, sem_b):
        wid = lax.axis_index("s") * NC + lax.axis_index("c")
        base = wid * bpw

        def load_idx(src):
            pltpu.sync_copy(src.at[pl.ds(base, bpw)], idx_v)

        def fire(tbl, buf, sem, off):
            def body(c, _):
                vec = idx_v[pl.ds(off + c * UNROLL, UNROLL)]
                for j in range(UNROLL):
                    i = c * UNROLL + j
                    pltpu.async_copy(tbl.at[pl.ds(vec[j], 1)],
                                     buf.at[pl.ds(i, 1)], sem)
                return 0
            lax.fori_loop(0, CH // UNROLL, body, 0)

        def drain(tbl, buf, sem, out_ref, off):
            pltpu.make_async_copy(tbl.at[pl.ds(0, CH)], buf, sem).wait()
            pltpu.sync_copy(buf, out_ref.at[pl.ds(base + off, CH)])

        def stage(tbl, out_ref):
            fire(tbl, rows_a, sem_a, 0)
            fire(tbl, rows_b, sem_b, CH)
            drain(tbl, rows_a, sem_a, out_ref, 0)
            drain(tbl, rows_b, sem_b, out_ref, CH)

        def stage_f(tbl, out_ref):
            fire(tbl, rows_fa, sem_a, 0)
            drain(tbl, rows_fa, sem_a, out_ref, 0)
            fire(tbl, rows_fa, sem_a, CH)
            drain(tbl, rows_fa, sem_a, out_ref, CH)

        load_idx(heads)
        stage(ent, hh_o)
        stage(ae, hx_o)
        load_idx(tails)
        stage(ent, tt_o)
        stage(ae, tx_o)
        load_idx(rels)
        stage_f(rel_t, r_o)
        stage_f(rinv_t, rinv_o)

    return sc_gather


def _tc_body(hh, tt, hx, tx, r, rinv, W_enc, b_enc, W_dec, b_dec,
             score_o, err_o):
    i = pl.program_id(0)
    W = W_enc[...]
    Wn = W * lax.rsqrt(jnp.sum(W * W, axis=1, keepdims=True))
    def unpack(x):
        u = lax.bitcast_convert_type(x, jnp.uint32)
        lo = lax.bitcast_convert_type(u << jnp.uint32(16), jnp.float32)
        hi = lax.bitcast_convert_type(u & jnp.uint32(0xFFFF0000), jnp.float32)
        return jnp.concatenate([lo, hi], axis=1)

    hxv = unpack(hx[...])
    txv = unpack(tx[...])
    be = b_enc[...]
    bd = b_dec[...]
    dn = (((1,), (1,)), ((), ()))
    hz = jnp.tanh(lax.dot_general(hxv, Wn, dn, preferred_element_type=jnp.float32) + be)
    tz = jnp.tanh(lax.dot_general(txv, Wn, dn, preferred_element_type=jnp.float32) + be)
    Wd = W_dec[...]
    hrec = lax.dot_general(hz, Wd, dn, preferred_element_type=jnp.float32) + bd
    trec = lax.dot_general(tz, Wd, dn, preferred_element_type=jnp.float32) + bd
    dh = hrec - hxv
    dt = trec - txv
    blk = jnp.sum(dh * dh) + jnp.sum(dt * dt)

    s1 = jnp.sum(unpack(hh[...]) * r[...] * unpack(tt[...]), axis=1)
    s2 = jnp.sum(tz * rinv[...] * hz, axis=1)
    score_o[...] = jnp.clip((s1 + s2) * 0.5, -20.0, 20.0)

    prev = jnp.where(i == 0, jnp.zeros((1, 1), jnp.float32), err_o[...])
    tot = prev + blk
    err_o[...] = jnp.where(i == GRID - 1, tot * (1.0 / (B * D)), tot)


def _tc_compute(hh, tt, hx, tx, r, rinv, W_enc, b_enc, W_dec, b_dec):
    row = pl.BlockSpec((BLK, D // 2), lambda i: (i, 0))
    rowf = pl.BlockSpec((BLK, D), lambda i: (i, 0))
    full = pl.BlockSpec((D, D), lambda i: (0, 0))
    vec = pl.BlockSpec((1, D), lambda i: (0, 0))
    return pl.pallas_call(
        _tc_body,
        grid=(GRID,),
        in_specs=[row, row, row, row, rowf, rowf, full, vec, full, vec],
        out_specs=[
            pl.BlockSpec((BLK,), lambda i: (i,)),
            pl.BlockSpec((1, 1), lambda i: (0, 0)),
        ],
        out_shape=[
            jax.ShapeDtypeStruct((B,), jnp.float32),
            jax.ShapeDtypeStruct((1, 1), jnp.float32),
        ],
    )(hh, tt, hx, tx, r, rinv, W_enc, b_enc, W_dec, b_dec)


def kernel(heads, rels, tails, ent_h_embs, rel_embs, rel_inv_embs,
           ae_emb, W_enc, b_enc, W_dec, b_dec):
    ent_row, ae_row = _tc_transpose(ent_h_embs.T, ae_emb.T)
    hh, tt, hx, tx, r, rinv = _make_sc_gather()(
        heads, tails, rels, ent_row, ae_row, rel_embs, rel_inv_embs)
    score, err = _tc_compute(
        hh, tt, hx, tx, r, rinv,
        W_enc, b_enc.reshape(1, D), W_dec, b_dec.reshape(1, D))
    return score, err.reshape(())


# speedup vs baseline: 1.6173x; 1.6173x over previous
import functools
import jax
import jax.numpy as jnp
from jax import lax
from jax.experimental import pallas as pl
from jax.experimental.pallas import tpu as pltpu
from jax.experimental.pallas import tpu_sc as plsc

B = 16384
D = 64
BLK = 2048
GRID = B // BLK

NV = 1000000
VCH = 8192
VGRID = (NV + VCH - 1) // VCH


def _tr_body(ent_t, ae_t, ent_o, ae_o):
    eye = jax.lax.broadcasted_iota(jnp.int32, (D, D), 0) == \
          jax.lax.broadcasted_iota(jnp.int32, (D, D), 1)
    ident = eye.astype(jnp.bfloat16)
    dn = (((0,), (0,)), ((), ()))

    def tr_pair(x):
        y = lax.dot_general(x.astype(jnp.bfloat16), ident, dn,
                            preferred_element_type=jnp.float32)
        return jnp.concatenate([y[: VCH // 2, :], y[VCH // 2:, :]], axis=1)

    ent_o[...] = tr_pair(ent_t[...])
    ae_o[...] = tr_pair(ae_t[...])


NROW = VGRID * VCH // 2


def _tc_transpose(ent_t, ae_t):
    col = pl.BlockSpec((D, VCH), lambda i: (0, i))
    row = pl.BlockSpec((VCH // 2, 2 * D), lambda i: (i, 0))
    return pl.pallas_call(
        _tr_body,
        grid=(VGRID,),
        in_specs=[col, col],
        out_specs=[row, row],
        out_shape=[
            jax.ShapeDtypeStruct((NROW, 2 * D), jnp.float32),
            jax.ShapeDtypeStruct((NROW, 2 * D), jnp.float32),
        ],
    )(ent_t, ae_t)


@functools.cache
def _make_sc_gather():
    info = plsc.get_sparse_core_info()
    NC, NS = info.num_cores, info.num_subcores
    NW = NC * NS
    bpw = B // NW
    UNROLL = 16
    CH = bpw // 2

    mesh = plsc.VectorSubcoreMesh(core_axis_name="c", subcore_axis_name="s")
    outb = jax.ShapeDtypeStruct((B, 2 * D), jnp.float32)
    outf = jax.ShapeDtypeStruct((B, D), jnp.float32)

    @functools.partial(
        pl.kernel,
        mesh=mesh,
        out_type=[outb] * 4 + [outf] * 2,
        scratch_types=[
            pltpu.VMEM((bpw,), jnp.int32),
            pltpu.VMEM((CH, 2 * D), jnp.float32),
            pltpu.VMEM((CH, 2 * D), jnp.float32),
            pltpu.VMEM((CH, D), jnp.float32),
            pltpu.SemaphoreType.DMA,
            pltpu.SemaphoreType.DMA,
        ],
    )
    def sc_gather(heads, tails, rels, ent, ae, rel_t, rinv_t,
                  hh_o, tt_o, hx_o, tx_o, r_o, rinv_o,
                  idx_v, rows_a, rows_b, rows_fa, sem_a, sem_b):
        wid = lax.axis_index("s") * NC + lax.axis_index("c")
        base = wid * bpw

        def load_idx(src):
            pltpu.sync_copy(src.at[pl.ds(base, bpw)], idx_v)

        def remap_idx():
            def body(k, _):
                v = idx_v[pl.ds(k * 16, 16)]
                g = lax.shift_left(lax.shift_right_logical(v, 13), 12) + \
                    lax.bitwise_and(v, 4095)
                idx_v[pl.ds(k * 16, 16)] = g
                return 0
            lax.fori_loop(0, bpw // 16, body, 0)

        def fire(tbl, buf, sem, off):
            def body(c, _):
                vec = idx_v[pl.ds(off + c * UNROLL, UNROLL)]
                for j in range(UNROLL):
                    i = c * UNROLL + j
                    pltpu.async_copy(tbl.at[pl.ds(vec[j], 1)],
                                     buf.at[pl.ds(i, 1)], sem)
                return 0
            lax.fori_loop(0, CH // UNROLL, body, 0)

        def drain(tbl, buf, sem, out_ref, off):
            pltpu.make_async_copy(tbl.at[pl.ds(0, CH)], buf, sem).wait()
            pltpu.sync_copy(buf, out_ref.at[pl.ds(base + off, CH)])

        def stage(tbl, out_ref):
            fire(tbl, rows_a, sem_a, 0)
            fire(tbl, rows_b, sem_b, CH)
            drain(tbl, rows_a, sem_a, out_ref, 0)
            drain(tbl, rows_b, sem_b, out_ref, CH)

        def stage_f(tbl, out_ref):
            fire(tbl, rows_fa, sem_a, 0)
            drain(tbl, rows_fa, sem_a, out_ref, 0)
            fire(tbl, rows_fa, sem_a, CH)
            drain(tbl, rows_fa, sem_a, out_ref, CH)

        load_idx(heads)
        remap_idx()
        stage(ent, hh_o)
        stage(ae, hx_o)
        load_idx(tails)
        remap_idx()
        stage(ent, tt_o)
        stage(ae, tx_o)
        load_idx(rels)
        stage_f(rel_t, r_o)
        stage_f(rinv_t, rinv_o)

    return sc_gather


def _tc_body(hh, tt, hx, tx, r, rinv, W_enc, b_enc, W_dec, b_dec, h_idx, t_idx,
             score_o, err_o):
    i = pl.program_id(0)
    W = W_enc[...]
    Wn = W * lax.rsqrt(jnp.sum(W * W, axis=1, keepdims=True))
    hsel = ((h_idx[...] >> 12) & 1).reshape(BLK, 1) == 1
    tsel = ((t_idx[...] >> 12) & 1).reshape(BLK, 1) == 1

    def unpack(x, sel):
        return jnp.where(sel, x[:, D:], x[:, :D])

    hxv = unpack(hx[...], hsel)
    txv = unpack(tx[...], tsel)
    be = b_enc[...]
    bd = b_dec[...]
    dn = (((1,), (1,)), ((), ()))
    hz = jnp.tanh(lax.dot_general(hxv, Wn, dn, preferred_element_type=jnp.float32) + be)
    tz = jnp.tanh(lax.dot_general(txv, Wn, dn, preferred_element_type=jnp.float32) + be)
    Wd = W_dec[...]
    hrec = lax.dot_general(hz, Wd, dn, preferred_element_type=jnp.float32) + bd
    trec = lax.dot_general(tz, Wd, dn, preferred_element_type=jnp.float32) + bd
    dh = hrec - hxv
    dt = trec - txv
    blk = jnp.sum(dh * dh) + jnp.sum(dt * dt)

    s1 = jnp.sum(unpack(hh[...], hsel) * r[...] * unpack(tt[...], tsel), axis=1)
    s2 = jnp.sum(tz * rinv[...] * hz, axis=1)
    score_o[...] = jnp.clip((s1 + s2) * 0.5, -20.0, 20.0)

    prev = jnp.where(i == 0, jnp.zeros((1, 1), jnp.float32), err_o[...])
    tot = prev + blk
    err_o[...] = jnp.where(i == GRID - 1, tot * (1.0 / (B * D)), tot)


def _tc_compute(hh, tt, hx, tx, r, rinv, W_enc, b_enc, W_dec, b_dec, heads, tails):
    row = pl.BlockSpec((BLK, 2 * D), lambda i: (i, 0))
    rowf = pl.BlockSpec((BLK, D), lambda i: (i, 0))
    full = pl.BlockSpec((D, D), lambda i: (0, 0))
    vec = pl.BlockSpec((1, D), lambda i: (0, 0))
    ivec = pl.BlockSpec((BLK,), lambda i: (i,))
    return pl.pallas_call(
        _tc_body,
        grid=(GRID,),
        in_specs=[row, row, row, row, rowf, rowf, full, vec, full, vec, ivec, ivec],
        out_specs=[
            pl.BlockSpec((BLK,), lambda i: (i,)),
            pl.BlockSpec((1, 1), lambda i: (0, 0)),
        ],
        out_shape=[
            jax.ShapeDtypeStruct((B,), jnp.float32),
            jax.ShapeDtypeStruct((1, 1), jnp.float32),
        ],
    )(hh, tt, hx, tx, r, rinv, W_enc, b_enc, W_dec, b_dec, heads, tails)


def kernel(heads, rels, tails, ent_h_embs, rel_embs, rel_inv_embs,
           ae_emb, W_enc, b_enc, W_dec, b_dec):
    ent_row, ae_row = _tc_transpose(ent_h_embs.T, ae_emb.T)
    hh, tt, hx, tx, r, rinv = _make_sc_gather()(
        heads, tails, rels, ent_row, ae_row, rel_embs, rel_inv_embs)
    score, err = _tc_compute(
        hh, tt, hx, tx, r, rinv,
        W_enc, b_enc.reshape(1, D), W_dec, b_dec.reshape(1, D), heads, tails)
    return score, err.reshape(())


# speedup vs baseline: 1.8690x; 1.1556x over previous
import functools
import jax
import jax.numpy as jnp
from jax import lax
from jax.experimental import pallas as pl
from jax.experimental.pallas import tpu as pltpu
from jax.experimental.pallas import tpu_sc as plsc

B = 16384
D = 64
BLK = 2048
GRID = B // BLK

NV = 1000000
VCH = 8192
VGRID = (NV + VCH - 1) // VCH


def _tr_body(ent_t, ae_t, ent_o, ae_o):
    eye = jax.lax.broadcasted_iota(jnp.int32, (D, D), 0) == \
          jax.lax.broadcasted_iota(jnp.int32, (D, D), 1)
    ident = eye.astype(jnp.bfloat16)
    dn = (((0,), (0,)), ((), ()))

    Q = VCH // 4

    def pk(a, b):
        ua = (lax.bitcast_convert_type(a, jnp.uint32) + jnp.uint32(0x8000)) >> jnp.uint32(16)
        ub = (lax.bitcast_convert_type(b, jnp.uint32) + jnp.uint32(0x8000)) & jnp.uint32(0xFFFF0000)
        return lax.bitcast_convert_type(ub | ua, jnp.int32)

    def tr_pack(x):
        y = lax.dot_general(x.astype(jnp.bfloat16), ident, dn,
                            preferred_element_type=jnp.float32)
        w01 = pk(y[:Q, :], y[Q:2 * Q, :])
        w23 = pk(y[2 * Q:3 * Q, :], y[3 * Q:, :])
        return jnp.concatenate([w01, w23], axis=1)

    ent_o[...] = tr_pack(ent_t[...])
    ae_o[...] = tr_pack(ae_t[...])


NROW = VGRID * VCH // 4


def _tc_transpose(ent_t, ae_t):
    col = pl.BlockSpec((D, VCH), lambda i: (0, i))
    row = pl.BlockSpec((VCH // 4, 2 * D), lambda i: (i, 0))
    return pl.pallas_call(
        _tr_body,
        grid=(VGRID,),
        in_specs=[col, col],
        out_specs=[row, row],
        out_shape=[
            jax.ShapeDtypeStruct((NROW, 2 * D), jnp.int32),
            jax.ShapeDtypeStruct((NROW, 2 * D), jnp.int32),
        ],
    )(ent_t, ae_t)


@functools.cache
def _make_sc_gather():
    info = plsc.get_sparse_core_info()
    NC, NS = info.num_cores, info.num_subcores
    NW = NC * NS
    bpw = B // NW
    UNROLL = 16
    CH = bpw // 2

    mesh = plsc.VectorSubcoreMesh(core_axis_name="c", subcore_axis_name="s")
    outb = jax.ShapeDtypeStruct((B, 2 * D), jnp.int32)
    outf = jax.ShapeDtypeStruct((B, D), jnp.float32)

    @functools.partial(
        pl.kernel,
        mesh=mesh,
        out_type=[outb] * 4 + [outf] * 2,
        scratch_types=[
            pltpu.VMEM((bpw,), jnp.int32),
            pltpu.VMEM((CH, 2 * D), jnp.int32),
            pltpu.VMEM((CH, 2 * D), jnp.int32),
            pltpu.VMEM((CH, D), jnp.float32),
            pltpu.SemaphoreType.DMA,
            pltpu.SemaphoreType.DMA,
        ],
    )
    def sc_gather(heads, tails, rels, ent, ae, rel_t, rinv_t,
                  hh_o, tt_o, hx_o, tx_o, r_o, rinv_o,
                  idx_v, rows_a, rows_b, rows_fa, sem_a, sem_b):
        wid = lax.axis_index("s") * NC + lax.axis_index("c")
        base = wid * bpw

        def load_idx(src):
            pltpu.sync_copy(src.at[pl.ds(base, bpw)], idx_v)

        def remap_idx():
            def body(k, _):
                v = idx_v[pl.ds(k * 16, 16)]
                g = lax.shift_left(lax.shift_right_logical(v, 13), 11) + \
                    lax.bitwise_and(v, 2047)
                idx_v[pl.ds(k * 16, 16)] = g
                return 0
            lax.fori_loop(0, bpw // 16, body, 0)

        def fire(tbl, buf, sem, off):
            def body(c, _):
                vec = idx_v[pl.ds(off + c * UNROLL, UNROLL)]
                for j in range(UNROLL):
                    i = c * UNROLL + j
                    pltpu.async_copy(tbl.at[pl.ds(vec[j], 1)],
                                     buf.at[pl.ds(i, 1)], sem)
                return 0
            lax.fori_loop(0, CH // UNROLL, body, 0)

        def drain(tbl, buf, sem, out_ref, off):
            pltpu.make_async_copy(tbl.at[pl.ds(0, CH)], buf, sem).wait()
            pltpu.sync_copy(buf, out_ref.at[pl.ds(base + off, CH)])

        def stage(tbl, out_ref):
            fire(tbl, rows_a, sem_a, 0)
            fire(tbl, rows_b, sem_b, CH)
            drain(tbl, rows_a, sem_a, out_ref, 0)
            drain(tbl, rows_b, sem_b, out_ref, CH)

        def stage_f(tbl, out_ref):
            fire(tbl, rows_fa, sem_a, 0)
            drain(tbl, rows_fa, sem_a, out_ref, 0)
            fire(tbl, rows_fa, sem_a, CH)
            drain(tbl, rows_fa, sem_a, out_ref, CH)

        load_idx(heads)
        remap_idx()
        stage(ent, hh_o)
        stage(ae, hx_o)
        load_idx(tails)
        remap_idx()
        stage(ent, tt_o)
        stage(ae, tx_o)
        load_idx(rels)
        stage_f(rel_t, r_o)
        stage_f(rinv_t, rinv_o)

    return sc_gather


def _tc_body(hh, tt, hx, tx, r, rinv, W_enc, b_enc, W_dec, b_dec, h_idx, t_idx,
             score_o, err_o):
    i = pl.program_id(0)
    W = W_enc[...]
    Wn = W * lax.rsqrt(jnp.sum(W * W, axis=1, keepdims=True))
    hsel = ((h_idx[...] >> 12) & 1).reshape(BLK, 1) == 1
    tsel = ((t_idx[...] >> 12) & 1).reshape(BLK, 1) == 1
    hhi = ((h_idx[...] >> 11) & 1).reshape(BLK, 1) == 1
    thi = ((t_idx[...] >> 11) & 1).reshape(BLK, 1) == 1

    def unpack(x, sel, hi):
        w = jnp.where(sel, x[:, D:], x[:, :D])
        u = lax.bitcast_convert_type(w, jnp.uint32)
        bits = jnp.where(hi, u & jnp.uint32(0xFFFF0000), u << jnp.uint32(16))
        return lax.bitcast_convert_type(bits, jnp.float32)

    hxv = unpack(hx[...], hsel, hhi)
    txv = unpack(tx[...], tsel, thi)
    be = b_enc[...]
    bd = b_dec[...]
    dn = (((1,), (1,)), ((), ()))
    hz = jnp.tanh(lax.dot_general(hxv, Wn, dn, preferred_element_type=jnp.float32) + be)
    tz = jnp.tanh(lax.dot_general(txv, Wn, dn, preferred_element_type=jnp.float32) + be)
    Wd = W_dec[...]
    hrec = lax.dot_general(hz, Wd, dn, preferred_element_type=jnp.float32) + bd
    trec = lax.dot_general(tz, Wd, dn, preferred_element_type=jnp.float32) + bd
    dh = hrec - hxv
    dt = trec - txv
    blk = jnp.sum(dh * dh) + jnp.sum(dt * dt)

    s1 = jnp.sum(unpack(hh[...], hsel, hhi) * r[...] * unpack(tt[...], tsel, thi), axis=1)
    s2 = jnp.sum(tz * rinv[...] * hz, axis=1)
    score_o[...] = jnp.clip((s1 + s2) * 0.5, -20.0, 20.0)

    prev = jnp.where(i == 0, jnp.zeros((1, 1), jnp.float32), err_o[...])
    tot = prev + blk
    err_o[...] = jnp.where(i == GRID - 1, tot * (1.0 / (B * D)), tot)


def _tc_compute(hh, tt, hx, tx, r, rinv, W_enc, b_enc, W_dec, b_dec, heads, tails):
    row = pl.BlockSpec((BLK, 2 * D), lambda i: (i, 0))
    rowf = pl.BlockSpec((BLK, D), lambda i: (i, 0))
    full = pl.BlockSpec((D, D), lambda i: (0, 0))
    vec = pl.BlockSpec((1, D), lambda i: (0, 0))
    ivec = pl.BlockSpec((BLK,), lambda i: (i,))
    return pl.pallas_call(
        _tc_body,
        grid=(GRID,),
        in_specs=[row, row, row, row, rowf, rowf, full, vec, full, vec, ivec, ivec],
        out_specs=[
            pl.BlockSpec((BLK,), lambda i: (i,)),
            pl.BlockSpec((1, 1), lambda i: (0, 0)),
        ],
        out_shape=[
            jax.ShapeDtypeStruct((B,), jnp.float32),
            jax.ShapeDtypeStruct((1, 1), jnp.float32),
        ],
    )(hh, tt, hx, tx, r, rinv, W_enc, b_enc, W_dec, b_dec, heads, tails)


def kernel(heads, rels, tails, ent_h_embs, rel_embs, rel_inv_embs,
           ae_emb, W_enc, b_enc, W_dec, b_dec):
    ent_row, ae_row = _tc_transpose(ent_h_embs.T, ae_emb.T)
    hh, tt, hx, tx, r, rinv = _make_sc_gather()(
        heads, tails, rels, ent_row, ae_row, rel_embs, rel_inv_embs)
    score, err = _tc_compute(
        hh, tt, hx, tx, r, rinv,
        W_enc, b_enc.reshape(1, D), W_dec, b_dec.reshape(1, D), heads, tails)
    return score, err.reshape(())
